# Initial kernel scaffold; baseline (speedup 1.0000x reference)
import jax
import jax.numpy as jnp
from jax.experimental import pallas as pl


def kernel(x, edge_index, batch, W_l, b_l, W_r, b_r, att_v2, bias_v2, gamma, beta, Wg1, att_src1, att_dst1, bias_g1, Wg2, att_src2, att_dst2, bias_g2, W1, b1, W2, b2):
    raise NotImplementedError("write your pallas kernel here")


# speedup vs baseline: 1.1166x; 1.1166x over previous
import jax
import jax.numpy as jnp
from jax.experimental import pallas as pl


def kernel(x, edge_index, batch, W_l, b_l, W_r, b_r, att_v2, bias_v2, gamma, beta, Wg1, att_src1, att_dst1, bias_g1, Wg2, att_src2, att_dst2, bias_g2, W1, b1, W2, b2):
    N = x.shape[0]
    G = 391
    NPG = 128
    HID = 64
    src, dst = edge_index[0], edge_index[1]
    xl = x @ W_l + b_l
    xr = x @ W_r + b_r
    h_e = jax.nn.leaky_relu(xl[src] + xr[dst], 0.2)
    w = jnp.exp(h_e @ att_v2)
    s = jax.ops.segment_sum(w, dst, num_segments=N)
    r = 1.0 / (s + 1e-16)
    out = jax.ops.segment_sum(xl[src] * (w * r[dst])[:, None], dst, num_segments=N)
    h = jax.nn.leaky_relu(out + bias_v2, 0.01)
    mean = h.mean()
    var = ((h - mean) ** 2).mean()
    h = gamma * ((h - mean) / (jnp.sqrt(var) + 1e-5)) + beta
    for (W, a_s, a_d, bias) in ((Wg1, att_src1, att_dst1, bias_g1),
                                (Wg2, att_src2, att_dst2, bias_g2)):
        xp = h @ W
        asrc = xp @ a_s
        adst = xp @ a_d
        w = jnp.exp(jax.nn.leaky_relu(asrc[src] + adst[dst], 0.2))
        wself = jnp.exp(jax.nn.leaky_relu(asrc + adst, 0.2))
        s = jax.ops.segment_sum(w, dst, num_segments=N) + wself
        r = 1.0 / (s + 1e-16)
        out = jax.ops.segment_sum(xp[src] * (w * r[dst])[:, None], dst, num_segments=N)
        out = out + (wself * r)[:, None] * xp
        h = jax.nn.relu(out + bias)
    feats = h.reshape(G, NPG * HID)
    z = jax.nn.leaky_relu(feats @ W1 + b1, 0.01)
    z = z @ W2 + b2
    return jax.nn.sigmoid(z)


# speedup vs baseline: 11.4463x; 10.2515x over previous
import functools

import jax
import jax.numpy as jnp
from jax import lax
from jax.experimental import pallas as pl
from jax.experimental.pallas import tpu as pltpu
from jax.experimental.pallas import tpu_sc as plsc

N = 50048
E = 800768
NROWS = E // 128
HID = 64
NSLICE = N // 16

_MESH = plsc.VectorSubcoreMesh(core_axis_name="c", subcore_axis_name="s")

_f32 = jnp.float32
_i32 = jnp.int32


def _wid():
    return lax.axis_index("s") * 2 + lax.axis_index("c")


def _chunk_range(wid):
    lo = wid * 195 + jnp.minimum(wid, 16)
    cnt = jnp.where(wid < 16, 196, 195)
    return lo, cnt


_IOTA16 = lambda: lax.broadcasted_iota(_i32, (16,), 0)


@functools.partial(
    pl.kernel,
    out_type=(jax.ShapeDtypeStruct((NROWS, 128), _f32),
              jax.ShapeDtypeStruct((32, N), _f32)),
    mesh=_MESH,
    compiler_params=pltpu.CompilerParams(needs_layout_passes=False, use_tc_tiling_on_sc=False),
    scratch_types=[
        pltpu.VMEM((128,), _i32),
        pltpu.VMEM((128,), _i32),
        pltpu.VMEM((128, HID), _f32),
        pltpu.VMEM((128, HID), _f32),
        pltpu.VMEM((128,), _f32),
        pltpu.VMEM((128,), _f32),
        pltpu.VMEM((N,), _f32),
        pltpu.SemaphoreType.DMA,
        pltpu.SemaphoreType.DMA,
    ],
)
def _k1(xl_hbm, xr_hbm, att_hbm, src_hbm, dst_hbm, zero1_hbm,
        w_out, s_out,
        src_iv, dst_iv, srows, drows, att_v, wbuf, s_part, sem1, sem2):
    wid = _wid()
    pltpu.sync_copy(att_hbm, att_v)
    pltpu.sync_copy(zero1_hbm, s_part)
    lo, cnt = _chunk_range(wid)
    iota16 = _IOTA16()

    def chunk(i, carry):
        row = lo + i
        pltpu.sync_copy(src_hbm.at[row], src_iv)
        pltpu.sync_copy(dst_hbm.at[row], dst_iv)
        cp1 = pltpu.async_copy(xl_hbm.at[src_iv], srows, sem1)
        cp2 = pltpu.async_copy(xr_hbm.at[dst_iv], drows, sem2)
        cp1.wait()
        cp2.wait()

        def dbody(d, accs):
            attb = plsc.load_gather(att_v, [iota16 * 0 + d])
            out = []
            for g in range(8):
                rid = iota16 + g * 16
                cid = iota16 * 0 + d
                s16 = plsc.load_gather(srows, [rid, cid])
                d16 = plsc.load_gather(drows, [rid, cid])
                za = s16 + d16
                lr = 0.6 * za + 0.4 * jnp.abs(za)
                out.append(accs[g] + lr * attb)
            return tuple(out)

        accs = lax.fori_loop(0, HID, dbody,
                             tuple(jnp.zeros((16,), _f32) for _ in range(8)))
        for g in range(8):
            w16 = jnp.exp(accs[g])
            wbuf[pl.ds(g * 16, 16)] = w16
            dst16 = dst_iv[pl.ds(g * 16, 16)]
            plsc.addupdate_scatter(s_part, [dst16], w16)
        pltpu.sync_copy(wbuf, w_out.at[row])
        return carry

    lax.fori_loop(0, cnt, chunk, 0)
    pltpu.sync_copy(s_part, s_out.at[wid])


@functools.partial(
    pl.kernel,
    out_type=jax.ShapeDtypeStruct((NROWS, 128), _f32),
    mesh=_MESH,
    compiler_params=pltpu.CompilerParams(needs_layout_passes=False, use_tc_tiling_on_sc=False),
    scratch_types=[
        pltpu.VMEM((128,), _i32),
        pltpu.VMEM((128,), _i32),
        pltpu.VMEM((N,), _f32),
        pltpu.VMEM((N,), _f32),
        pltpu.VMEM((128,), _f32),
    ],
)
def _k3(asrc_hbm, adst_hbm, src_hbm, dst_hbm,
        w_out,
        src_iv, dst_iv, asv, adv, wbuf):
    wid = _wid()
    pltpu.sync_copy(asrc_hbm, asv)
    pltpu.sync_copy(adst_hbm, adv)
    lo, cnt = _chunk_range(wid)

    def chunk(i, carry):
        row = lo + i
        pltpu.sync_copy(src_hbm.at[row], src_iv)
        pltpu.sync_copy(dst_hbm.at[row], dst_iv)
        for g in range(8):
            src16 = src_iv[pl.ds(g * 16, 16)]
            dst16 = dst_iv[pl.ds(g * 16, 16)]
            a16 = plsc.load_gather(asv, [src16])
            b16 = plsc.load_gather(adv, [dst16])
            z = a16 + b16
            lr = 0.6 * z + 0.4 * jnp.abs(z)
            wbuf[pl.ds(g * 16, 16)] = jnp.exp(lr)
        pltpu.sync_copy(wbuf, w_out.at[row])
        return carry

    lax.fori_loop(0, cnt, chunk, 0)


@functools.partial(
    pl.kernel,
    out_type=jax.ShapeDtypeStruct((32, N), _f32),
    mesh=_MESH,
    compiler_params=pltpu.CompilerParams(needs_layout_passes=False, use_tc_tiling_on_sc=False),
    scratch_types=[
        pltpu.VMEM((128,), _i32),
        pltpu.VMEM((128,), _f32),
        pltpu.VMEM((N,), _f32),
    ],
)
def _ks(dst_hbm, w_hbm, zero1_hbm,
        s_out,
        dst_iv, w_iv, s_part):
    wid = _wid()
    pltpu.sync_copy(zero1_hbm, s_part)
    lo, cnt = _chunk_range(wid)

    def chunk(i, carry):
        row = lo + i
        pltpu.sync_copy(dst_hbm.at[row], dst_iv)
        pltpu.sync_copy(w_hbm.at[row], w_iv)
        for g in range(8):
            dst16 = dst_iv[pl.ds(g * 16, 16)]
            w16 = w_iv[pl.ds(g * 16, 16)]
            plsc.addupdate_scatter(s_part, [dst16], w16)
        return carry

    lax.fori_loop(0, cnt, chunk, 0)
    pltpu.sync_copy(s_part, s_out.at[wid])


@functools.partial(
    pl.kernel,
    out_type=jax.ShapeDtypeStruct((NROWS, 128), _f32),
    mesh=_MESH,
    compiler_params=pltpu.CompilerParams(needs_layout_passes=False, use_tc_tiling_on_sc=False),
    scratch_types=[
        pltpu.VMEM((128,), _i32),
        pltpu.VMEM((128,), _f32),
        pltpu.VMEM((N,), _f32),
        pltpu.VMEM((128,), _f32),
    ],
)
def _k4(dst_hbm, w_hbm, r_hbm,
        a_out,
        dst_iv, w_iv, r_v, abuf):
    wid = _wid()
    pltpu.sync_copy(r_hbm, r_v)
    lo, cnt = _chunk_range(wid)

    def chunk(i, carry):
        row = lo + i
        pltpu.sync_copy(dst_hbm.at[row], dst_iv)
        pltpu.sync_copy(w_hbm.at[row], w_iv)
        for g in range(8):
            dst16 = dst_iv[pl.ds(g * 16, 16)]
            w16 = w_iv[pl.ds(g * 16, 16)]
            r16 = plsc.load_gather(r_v, [dst16])
            abuf[pl.ds(g * 16, 16)] = w16 * r16
        pltpu.sync_copy(abuf, a_out.at[row])
        return carry

    lax.fori_loop(0, cnt, chunk, 0)


@functools.partial(
    pl.kernel,
    out_type=jax.ShapeDtypeStruct((2, N, 32), _f32),
    mesh=_MESH,
    compiler_params=pltpu.CompilerParams(needs_layout_passes=False, use_tc_tiling_on_sc=False),
    scratch_types=[
        pltpu.VMEM((128,), _i32),
        pltpu.VMEM((128,), _i32),
        pltpu.VMEM((128,), _f32),
        pltpu.VMEM((128, 32), _f32),
        pltpu.VMEM((128, 32), _f32),
        pltpu.VMEM_SHARED((N, 32), _f32),
        pltpu.SemaphoreType.DMA,
    ],
)
def _k2(t0_hbm, t1_hbm, src_hbm, dst_hbm, a_hbm, zero2_hbm,
        out_hbm,
        src_iv, dst_iv, a_iv, rows_v, sbuf, acc, sem):
    cid = lax.axis_index("c")
    tid = lax.axis_index("s")
    pltpu.sync_copy(zero2_hbm.at[pl.ds(tid * NSLICE, NSLICE)],
                    acc.at[pl.ds(tid * NSLICE, NSLICE)])
    plsc.subcore_barrier()
    lo = tid * (NROWS // 16)
    bidx = [jnp.full((16, 1), l, dtype=_i32) for l in range(16)]
    _dn = lax.GatherDimensionNumbers(offset_dims=(), collapsed_slice_dims=(0,),
                                     start_index_map=(0,))

    def _bcast_lane(vec, l):
        return lax.gather(vec, bidx[l], dimension_numbers=_dn,
                          slice_sizes=(1,),
                          mode=lax.GatherScatterMode.PROMISE_IN_BOUNDS)

    def run(tbl, outslice):
        def chunk(i, carry):
            row = lo + i
            pltpu.sync_copy(src_hbm.at[row], src_iv)
            pltpu.sync_copy(dst_hbm.at[row], dst_iv)
            pltpu.sync_copy(a_hbm.at[row], a_iv)
            pltpu.async_copy(tbl.at[src_iv], rows_v, sem).wait()
            for g in range(8):
                alpha = a_iv[pl.ds(g * 16, 16)]
                for l in range(16):
                    j = g * 16 + l
                    ab = _bcast_lane(alpha, l)
                    sbuf[j, pl.ds(0, 16)] = rows_v[j, pl.ds(0, 16)] * ab
                    sbuf[j, pl.ds(16, 16)] = rows_v[j, pl.ds(16, 16)] * ab
            pltpu.sync_copy(sbuf, acc.at[dst_iv], add=True)
            return carry

        lax.fori_loop(0, NROWS // 16, chunk, 0)
        plsc.subcore_barrier()
        pltpu.sync_copy(acc.at[pl.ds(tid * NSLICE, NSLICE)],
                        outslice.at[pl.ds(tid * NSLICE, NSLICE)])

    @pl.when(cid == 0)
    def _():
        run(t0_hbm, out_hbm.at[0])

    @pl.when(cid == 1)
    def _():
        run(t1_hbm, out_hbm.at[1])


def _lrelu(x, s):
    return jnp.where(x > 0, x, s * x)


def kernel(x, edge_index, batch, W_l, b_l, W_r, b_r, att_v2, bias_v2, gamma,
           beta, Wg1, att_src1, att_dst1, bias_g1, Wg2, att_src2, att_dst2,
           bias_g2, W1, b1, W2, b2):
    src2d = edge_index[0].reshape(NROWS, 128)
    dst2d = edge_index[1].reshape(NROWS, 128)
    zero1 = jnp.zeros((N,), _f32)
    zero2 = jnp.zeros((N, 32), _f32)

    xl = x @ W_l + b_l
    xr = x @ W_r + b_r
    att_pad = jnp.concatenate([att_v2, jnp.zeros((64,), _f32)])
    w1, s1p = _k1(xl, xr, att_pad, src2d, dst2d, zero1)
    s1 = s1p.sum(axis=0)
    r1 = 1.0 / (s1 + 1e-16)
    a1 = _k4(dst2d, w1, r1)
    o1 = _k2(xl[:, :32], xl[:, 32:], src2d, dst2d, a1, zero2)
    out1 = jnp.concatenate([o1[0], o1[1]], axis=1)
    h = _lrelu(out1 + bias_v2, 0.01)

    mean = h.mean()
    var = ((h - mean) ** 2).mean()
    inv = 1.0 / (jnp.sqrt(var) + 1e-5)
    a_vec = inv * gamma
    b_vec = beta - mean * inv * gamma

    for li, (W, a_s, a_d, bias) in enumerate((
            (Wg1, att_src1, att_dst1, bias_g1),
            (Wg2, att_src2, att_dst2, bias_g2))):
        if li == 0:
            W_eff = a_vec[:, None] * W
            b_eff = b_vec @ W
        else:
            W_eff = W
            b_eff = jnp.zeros((HID,), _f32)
        xp = h @ W_eff + b_eff
        asv = xp @ a_s
        adv = xp @ a_d
        w_e = _k3(asv, adv, src2d, dst2d)
        sp = _ks(dst2d, w_e, zero1)
        wself = jnp.exp(_lrelu(asv + adv, 0.2))
        s = sp.sum(axis=0) + wself
        r = 1.0 / (s + 1e-16)
        a_e = _k4(dst2d, w_e, r)
        o = _k2(xp[:, :32], xp[:, 32:], src2d, dst2d, a_e, zero2)
        outf = jnp.concatenate([o[0], o[1]], axis=1) + (wself * r)[:, None] * xp
        h = jax.nn.relu(outf + bias)

    feats = h.reshape(391, 128 * HID)
    z = _lrelu(feats @ W1 + b1, 0.01)
    z = z @ W2 + b2
    return jax.nn.sigmoid(z)
